# baseline (speedup 1.0000x reference)
import functools

import jax
import jax.numpy as jnp
from jax import lax
from jax.experimental import pallas as pl
from jax.experimental.pallas import tpu as pltpu
from jax.experimental.pallas import tpu_sc as plsc

DIST_DIM = 128
NODE_DIM = 256
N_POINTS = 4096
N_BATCH = 4
BIN_SIZE = 128
N_BINS = 32
N_KEYS = 64
CHUNK = 128
PAY_W = 128
DIST_MULT = 0.1


def _rank_body(x_ref, w_ref, m_ref, rank_ref):
    x = x_ref[0]
    w = w_ref[...]
    mk = m_ref[0]

    mul = lax.dot_general(x, w, (((1,), (0,)), ((), ())),
                          preferred_element_type=jnp.float32)
    cmul = jnp.concatenate([mul, -mul], axis=1)
    mx = jnp.max(cmul, axis=1, keepdims=True)
    lane = lax.broadcasted_iota(jnp.int32, (N_POINTS, N_BINS), 1).astype(jnp.float32)
    amax = jnp.min(jnp.where(cmul == mx, lane, float(N_KEYS)),
                   axis=1, keepdims=True)
    binf = amax + (N_BINS - 1) * (1.0 - mk)

    keylane = lax.broadcasted_iota(jnp.int32, (N_POINTS, N_KEYS), 1).astype(jnp.float32)
    onehot = jnp.where(binf == keylane, 1.0, 0.0)

    ones_col = jnp.ones((N_POINTS, 1), jnp.float32)
    total = lax.dot_general(ones_col, onehot, (((0,), (0,)), ((), ())))
    r64 = lax.broadcasted_iota(jnp.int32, (N_KEYS, N_KEYS), 0).astype(jnp.float32)
    c64 = lax.broadcasted_iota(jnp.int32, (N_KEYS, N_KEYS), 1).astype(jnp.float32)
    strict_upper = jnp.where(r64 < c64, 1.0, 0.0)
    offset = lax.dot_general(total, strict_upper, (((1,), (0,)), ((), ())))

    rr = lax.broadcasted_iota(jnp.int32, (CHUNK, CHUNK), 0).astype(jnp.float32)
    cc = lax.broadcasted_iota(jnp.int32, (CHUNK, CHUNK), 1).astype(jnp.float32)
    strict_lower = jnp.where(cc < rr, 1.0, 0.0)
    ones_row = jnp.ones((1, CHUNK), jnp.float32)

    base = offset
    for blk in range(N_POINTS // CHUNK):
        ob = onehot[blk * CHUNK:(blk + 1) * CHUNK]
        pre = lax.dot_general(strict_lower, ob, (((1,), (0,)), ((), ()))) + base
        rk = jnp.sum(ob * pre, axis=1, keepdims=True)
        rank_ref[0, blk * CHUNK:(blk + 1) * CHUNK, :] = rk.astype(jnp.int32)
        base = base + lax.dot_general(ones_row, ob, (((1,), (0,)), ((), ())))


def _dm_body(x_ref, m_ref, o_ref):
    x = x_ref[0]
    m = m_ref[0]
    xm = x * m
    na = jnp.sum(xm * xm, axis=1, keepdims=True)
    ri = lax.broadcasted_iota(jnp.int32, (BIN_SIZE, BIN_SIZE), 0).astype(jnp.float32)
    ci = lax.broadcasted_iota(jnp.int32, (BIN_SIZE, BIN_SIZE), 1).astype(jnp.float32)
    ident = jnp.where(ri == ci, 1.0, 0.0)
    nb = lax.dot_general(na, ident, (((0,), (0,)), ((), ())))
    mrow = lax.dot_general(m, ident, (((0,), (0,)), ((), ())))
    g = lax.dot_general(xm, xm, (((1,), (1,)), ((), ())),
                        preferred_element_type=jnp.float32)
    d = jnp.sqrt(jnp.maximum(na - 2.0 * g + nb, 1e-6))
    e = jnp.minimum(jnp.exp(-DIST_MULT * d), 1.0)
    e = jnp.maximum(e, 0.0)
    o_ref[0] = e * m * mrow


@functools.lru_cache(maxsize=1)
def _make_sc_scatter():
    info = plsc.get_sparse_core_info()
    nc = info.num_cores
    mesh = plsc.VectorSubcoreMesh(core_axis_name="c", subcore_axis_name="s")
    out_type = [
        jax.ShapeDtypeStruct((N_BATCH, N_POINTS, PAY_W), jnp.int32),
        jax.ShapeDtypeStruct((N_BATCH, N_POINTS, DIST_DIM), jnp.float32),
        jax.ShapeDtypeStruct((N_BATCH, N_POINTS, NODE_DIM), jnp.float32),
    ]
    scratch = [
        pltpu.VMEM((CHUNK,), jnp.int32),
        pltpu.VMEM((CHUNK, PAY_W), jnp.int32),
        pltpu.VMEM((CHUNK, DIST_DIM), jnp.float32),
        pltpu.VMEM((CHUNK, NODE_DIM), jnp.float32),
        pltpu.SemaphoreType.DMA,
    ]

    @functools.partial(pl.kernel, mesh=mesh, out_type=out_type,
                       scratch_types=scratch)
    def scatter_k(rank_hbm, pay_hbm, xmsg_hbm, xnode_hbm,
                  pay_out, xmsgb_out, xnodeb_out,
                  idx_v, pay_v, xm_v, xn_v, sem):
        wid = lax.axis_index("s") * nc + lax.axis_index("c")
        base = wid * CHUNK
        for b in range(N_BATCH):
            pltpu.sync_copy(rank_hbm.at[b].at[pl.ds(base, CHUNK)], idx_v)
            pltpu.sync_copy(pay_hbm.at[b].at[pl.ds(base, CHUNK), :], pay_v)
            pltpu.sync_copy(xmsg_hbm.at[b].at[pl.ds(base, CHUNK), :], xm_v)
            pltpu.sync_copy(xnode_hbm.at[b].at[pl.ds(base, CHUNK), :], xn_v)
            c1 = pltpu.async_copy(pay_v, pay_out.at[b].at[idx_v], sem)
            c2 = pltpu.async_copy(xm_v, xmsgb_out.at[b].at[idx_v], sem)
            c3 = pltpu.async_copy(xn_v, xnodeb_out.at[b].at[idx_v], sem)
            c1.wait()
            c2.wait()
            c3.wait()

    return scatter_k


def _rank_call(x_msg, w16, mskf_col):
    return pl.pallas_call(
        _rank_body,
        grid=(N_BATCH,),
        in_specs=[
            pl.BlockSpec((1, N_POINTS, DIST_DIM), lambda b: (b, 0, 0)),
            pl.BlockSpec((DIST_DIM, N_BINS // 2), lambda b: (0, 0)),
            pl.BlockSpec((1, N_POINTS, 1), lambda b: (b, 0, 0)),
        ],
        out_specs=pl.BlockSpec((1, N_POINTS, 1), lambda b: (b, 0, 0)),
        out_shape=jax.ShapeDtypeStruct((N_BATCH, N_POINTS, 1), jnp.int32),
    )(x_msg, w16, mskf_col)


def _dm_call(xb, mb):
    return pl.pallas_call(
        _dm_body,
        grid=(N_BATCH * N_BINS,),
        in_specs=[
            pl.BlockSpec((1, BIN_SIZE, DIST_DIM), lambda i: (i, 0, 0)),
            pl.BlockSpec((1, BIN_SIZE, 1), lambda i: (i, 0, 0)),
        ],
        out_specs=pl.BlockSpec((1, BIN_SIZE, BIN_SIZE), lambda i: (i, 0, 0)),
        out_shape=jax.ShapeDtypeStruct(
            (N_BATCH * N_BINS, BIN_SIZE, BIN_SIZE), jnp.float32),
    )(xb, mb)


@jax.jit
def kernel(x_msg, x_node, msk, W):
    mskf = msk.astype(jnp.float32)
    w16 = W[:, : N_BINS // 2]

    rank = _rank_call(x_msg, w16, mskf[..., None])
    rank2 = rank.reshape(N_BATCH, N_POINTS)

    packed = 2 * jnp.arange(N_POINTS, dtype=jnp.int32)[None, :] + msk.astype(
        jnp.int32)
    pay = jnp.broadcast_to(packed[..., None], (N_BATCH, N_POINTS, PAY_W))

    pay_out, xmsgb, xnodeb = _make_sc_scatter()(rank2, pay, x_msg, x_node)

    packed_sorted = pay_out[..., 0]
    mskb_flat = (packed_sorted & 1).astype(jnp.float32)
    xb = xmsgb.reshape(N_BATCH * N_BINS, BIN_SIZE, DIST_DIM)
    mb = mskb_flat.reshape(N_BATCH * N_BINS, BIN_SIZE, 1)
    dm = _dm_call(xb, mb)

    bins_split = (packed_sorted >> 1).reshape(N_BATCH, N_BINS, BIN_SIZE)
    x_features_binned = xnodeb.reshape(N_BATCH, N_BINS, BIN_SIZE, NODE_DIM)
    dm_out = dm.reshape(N_BATCH, N_BINS, BIN_SIZE, BIN_SIZE, 1)
    msk_f_binned = mskb_flat.reshape(N_BATCH, N_BINS, BIN_SIZE, 1)
    return (bins_split, x_features_binned, dm_out, msk_f_binned)


# speedup vs baseline: 1.0586x; 1.0586x over previous
import functools

import jax
import jax.numpy as jnp
from jax import lax
from jax.experimental import pallas as pl
from jax.experimental.pallas import tpu as pltpu
from jax.experimental.pallas import tpu_sc as plsc

DIST_DIM = 128
NODE_DIM = 256
N_POINTS = 4096
N_BATCH = 4
BIN_SIZE = 128
N_BINS = 32
N_KEYS = 64
CHUNK = 128
PAY_W = 128
DIST_MULT = 0.1


def _rank_body(x_ref, w_ref, m_ref, rank_ref):
    x = x_ref[0]
    w = w_ref[...]
    mk = m_ref[0]

    mul = lax.dot_general(x, w, (((1,), (0,)), ((), ())),
                          preferred_element_type=jnp.float32)
    cmul = jnp.concatenate([mul, -mul], axis=1)
    mx = jnp.max(cmul, axis=1, keepdims=True)
    lane = lax.broadcasted_iota(jnp.int32, (N_POINTS, N_BINS), 1).astype(jnp.float32)
    amax = jnp.min(jnp.where(cmul == mx, lane, float(N_KEYS)),
                   axis=1, keepdims=True)
    binf = amax + (N_BINS - 1) * (1.0 - mk)

    keylane = lax.broadcasted_iota(jnp.int32, (N_POINTS, N_KEYS), 1).astype(jnp.float32)
    onehot = jnp.where(binf == keylane, 1.0, 0.0)

    ones_col = jnp.ones((N_POINTS, 1), jnp.float32)
    total = lax.dot_general(ones_col, onehot, (((0,), (0,)), ((), ())))
    r64 = lax.broadcasted_iota(jnp.int32, (N_KEYS, N_KEYS), 0).astype(jnp.float32)
    c64 = lax.broadcasted_iota(jnp.int32, (N_KEYS, N_KEYS), 1).astype(jnp.float32)
    strict_upper = jnp.where(r64 < c64, 1.0, 0.0)
    offset = lax.dot_general(total, strict_upper, (((1,), (0,)), ((), ())))

    rr = lax.broadcasted_iota(jnp.int32, (CHUNK, CHUNK), 0).astype(jnp.float32)
    cc = lax.broadcasted_iota(jnp.int32, (CHUNK, CHUNK), 1).astype(jnp.float32)
    strict_lower = jnp.where(cc < rr, 1.0, 0.0)
    ones_row = jnp.ones((1, CHUNK), jnp.float32)

    base = offset
    for blk in range(N_POINTS // CHUNK):
        ob = onehot[blk * CHUNK:(blk + 1) * CHUNK]
        pre = lax.dot_general(strict_lower, ob, (((1,), (0,)), ((), ()))) + base
        rk = jnp.sum(ob * pre, axis=1, keepdims=True)
        rank_ref[0, blk * CHUNK:(blk + 1) * CHUNK, :] = rk.astype(jnp.int32)
        base = base + lax.dot_general(ones_row, ob, (((1,), (0,)), ((), ())))


def _dm_body(x_ref, m_ref, id_ref, o_ref):
    x = x_ref[0]
    m = m_ref[0]
    ident = id_ref[...]
    xm = x * m
    na = jnp.sum(xm * xm, axis=1, keepdims=True)
    nb = lax.dot_general(na, ident, (((0,), (0,)), ((), ())))
    mrow = lax.dot_general(m, ident, (((0,), (0,)), ((), ())))
    g = lax.dot_general(xm, xm, (((1,), (1,)), ((), ())),
                        preferred_element_type=jnp.float32)
    d = jnp.sqrt(jnp.maximum(na - 2.0 * g + nb, 1e-6))
    e = jnp.minimum(jnp.exp(-DIST_MULT * d), 1.0)
    e = jnp.maximum(e, 0.0)
    o_ref[0] = e * m * mrow


@functools.lru_cache(maxsize=1)
def _make_sc_scatter_a():
    info = plsc.get_sparse_core_info()
    nc = info.num_cores
    mesh = plsc.VectorSubcoreMesh(core_axis_name="c", subcore_axis_name="s")
    out_type = [
        jax.ShapeDtypeStruct((N_BATCH, N_POINTS, PAY_W), jnp.int32),
        jax.ShapeDtypeStruct((N_BATCH, N_POINTS, DIST_DIM), jnp.float32),
    ]
    scratch = [
        pltpu.VMEM((CHUNK,), jnp.int32),
        pltpu.VMEM((CHUNK, PAY_W), jnp.int32),
        pltpu.VMEM((CHUNK, DIST_DIM), jnp.float32),
        pltpu.SemaphoreType.DMA,
    ]

    @functools.partial(pl.kernel, mesh=mesh, out_type=out_type,
                       scratch_types=scratch)
    def scatter_a(rank_hbm, pay_hbm, xmsg_hbm,
                  pay_out, xmsgb_out,
                  idx_v, pay_v, xm_v, sem):
        wid = lax.axis_index("s") * nc + lax.axis_index("c")
        base = wid * CHUNK
        for b in range(N_BATCH):
            pltpu.sync_copy(rank_hbm.at[b].at[pl.ds(base, CHUNK)], idx_v)
            pltpu.sync_copy(pay_hbm.at[b].at[pl.ds(base, CHUNK), :], pay_v)
            pltpu.sync_copy(xmsg_hbm.at[b].at[pl.ds(base, CHUNK), :], xm_v)
            c1 = pltpu.async_copy(pay_v, pay_out.at[b].at[idx_v], sem)
            c2 = pltpu.async_copy(xm_v, xmsgb_out.at[b].at[idx_v], sem)
            c1.wait()
            c2.wait()

    return scatter_a


@functools.lru_cache(maxsize=1)
def _make_sc_scatter_b():
    info = plsc.get_sparse_core_info()
    nc = info.num_cores
    mesh = plsc.VectorSubcoreMesh(core_axis_name="c", subcore_axis_name="s")
    out_type = jax.ShapeDtypeStruct((N_BATCH, N_POINTS, NODE_DIM), jnp.float32)
    scratch = [
        pltpu.VMEM((CHUNK,), jnp.int32),
        pltpu.VMEM((CHUNK, NODE_DIM), jnp.float32),
        pltpu.SemaphoreType.DMA,
    ]

    @functools.partial(pl.kernel, mesh=mesh, out_type=out_type,
                       scratch_types=scratch)
    def scatter_b(rank_hbm, xnode_hbm, xnodeb_out, idx_v, xn_v, sem):
        wid = lax.axis_index("s") * nc + lax.axis_index("c")
        base = wid * CHUNK
        for b in range(N_BATCH):
            pltpu.sync_copy(rank_hbm.at[b].at[pl.ds(base, CHUNK)], idx_v)
            pltpu.sync_copy(xnode_hbm.at[b].at[pl.ds(base, CHUNK), :], xn_v)
            pltpu.async_copy(xn_v, xnodeb_out.at[b].at[idx_v], sem).wait()

    return scatter_b


def _rank_call(x_msg, w16, mskf_col):
    return pl.pallas_call(
        _rank_body,
        grid=(N_BATCH,),
        in_specs=[
            pl.BlockSpec((1, N_POINTS, DIST_DIM), lambda b: (b, 0, 0)),
            pl.BlockSpec((DIST_DIM, N_BINS // 2), lambda b: (0, 0)),
            pl.BlockSpec((1, N_POINTS, 1), lambda b: (b, 0, 0)),
        ],
        out_specs=pl.BlockSpec((1, N_POINTS, 1), lambda b: (b, 0, 0)),
        out_shape=jax.ShapeDtypeStruct((N_BATCH, N_POINTS, 1), jnp.int32),
    )(x_msg, w16, mskf_col)


def _dm_call(xb, mb, ident):
    return pl.pallas_call(
        _dm_body,
        grid=(N_BATCH * N_BINS,),
        in_specs=[
            pl.BlockSpec((1, BIN_SIZE, DIST_DIM), lambda i: (i, 0, 0)),
            pl.BlockSpec((1, BIN_SIZE, 1), lambda i: (i, 0, 0)),
            pl.BlockSpec((BIN_SIZE, BIN_SIZE), lambda i: (0, 0)),
        ],
        out_specs=pl.BlockSpec((1, BIN_SIZE, BIN_SIZE), lambda i: (i, 0, 0)),
        out_shape=jax.ShapeDtypeStruct(
            (N_BATCH * N_BINS, BIN_SIZE, BIN_SIZE), jnp.float32),
    )(xb, mb, ident)


@jax.jit
def kernel(x_msg, x_node, msk, W):
    mskf = msk.astype(jnp.float32)
    w16 = W[:, : N_BINS // 2]

    rank = _rank_call(x_msg, w16, mskf[..., None])
    rank2 = rank.reshape(N_BATCH, N_POINTS)

    packed = 2 * jnp.arange(N_POINTS, dtype=jnp.int32)[None, :] + msk.astype(
        jnp.int32)
    pay = jnp.broadcast_to(packed[..., None], (N_BATCH, N_POINTS, PAY_W))

    pay_out, xmsgb = _make_sc_scatter_a()(rank2, pay, x_msg)
    xnodeb = _make_sc_scatter_b()(rank2, x_node)

    packed_sorted = pay_out[..., 0]
    mskb_flat = (packed_sorted & 1).astype(jnp.float32)
    xb = xmsgb.reshape(N_BATCH * N_BINS, BIN_SIZE, DIST_DIM)
    mb = mskb_flat.reshape(N_BATCH * N_BINS, BIN_SIZE, 1)
    ident = jnp.eye(BIN_SIZE, dtype=jnp.float32)
    dm = _dm_call(xb, mb, ident)

    bins_split = (packed_sorted >> 1).reshape(N_BATCH, N_BINS, BIN_SIZE)
    x_features_binned = xnodeb.reshape(N_BATCH, N_BINS, BIN_SIZE, NODE_DIM)
    dm_out = dm.reshape(N_BATCH, N_BINS, BIN_SIZE, BIN_SIZE, 1)
    msk_f_binned = mskb_flat.reshape(N_BATCH, N_BINS, BIN_SIZE, 1)
    return (bins_split, x_features_binned, dm_out, msk_f_binned)


# speedup vs baseline: 1.0628x; 1.0040x over previous
import functools

import jax
import jax.numpy as jnp
from jax import lax
from jax.experimental import pallas as pl
from jax.experimental.pallas import tpu as pltpu
from jax.experimental.pallas import tpu_sc as plsc

DIST_DIM = 128
NODE_DIM = 256
N_POINTS = 4096
N_BATCH = 4
BIN_SIZE = 128
N_BINS = 32
N_KEYS = 64
CHUNK = 128
PAY_W = 128
DIST_MULT = 0.1


def _rank_body(x_ref, w_ref, m_ref, rank_ref):
    x = x_ref[0]
    w = w_ref[...]
    mk = m_ref[0]

    mul = lax.dot_general(x, w, (((1,), (0,)), ((), ())),
                          preferred_element_type=jnp.float32)
    cmul = jnp.concatenate([mul, -mul], axis=1)
    mx = jnp.max(cmul, axis=1, keepdims=True)
    lane = lax.broadcasted_iota(jnp.int32, (N_POINTS, N_BINS), 1).astype(jnp.float32)
    amax = jnp.min(jnp.where(cmul == mx, lane, float(N_KEYS)),
                   axis=1, keepdims=True)
    binf = amax + (N_BINS - 1) * (1.0 - mk)

    keylane = lax.broadcasted_iota(jnp.int32, (N_POINTS, N_KEYS), 1).astype(jnp.float32)
    onehot = jnp.where(binf == keylane, 1.0, 0.0)

    ones_col = jnp.ones((N_POINTS, 1), jnp.float32)
    total = lax.dot_general(ones_col, onehot, (((0,), (0,)), ((), ())))
    r64 = lax.broadcasted_iota(jnp.int32, (N_KEYS, N_KEYS), 0).astype(jnp.float32)
    c64 = lax.broadcasted_iota(jnp.int32, (N_KEYS, N_KEYS), 1).astype(jnp.float32)
    strict_upper = jnp.where(r64 < c64, 1.0, 0.0)
    offset = lax.dot_general(total, strict_upper, (((1,), (0,)), ((), ())))

    rr = lax.broadcasted_iota(jnp.int32, (CHUNK, CHUNK), 0).astype(jnp.float32)
    cc = lax.broadcasted_iota(jnp.int32, (CHUNK, CHUNK), 1).astype(jnp.float32)
    strict_lower = jnp.where(cc < rr, 1.0, 0.0)
    ones_row = jnp.ones((1, CHUNK), jnp.float32)

    base = offset
    for blk in range(N_POINTS // CHUNK):
        ob = onehot[blk * CHUNK:(blk + 1) * CHUNK]
        pre = lax.dot_general(strict_lower, ob, (((1,), (0,)), ((), ()))) + base
        rk = jnp.sum(ob * pre, axis=1, keepdims=True)
        rank_ref[0, blk * CHUNK:(blk + 1) * CHUNK, :] = rk.astype(jnp.int32)
        base = base + lax.dot_general(ones_row, ob, (((1,), (0,)), ((), ())))


def _dm_body(x_ref, m_ref, id_ref, o_ref):
    x = x_ref[0]
    m = m_ref[0]
    ident = id_ref[...]
    xm = x * m
    na = jnp.sum(xm * xm, axis=1, keepdims=True)
    nb = lax.dot_general(na, ident, (((0,), (0,)), ((), ())))
    mrow = lax.dot_general(m, ident, (((0,), (0,)), ((), ())))
    g = lax.dot_general(xm, xm, (((1,), (1,)), ((), ())),
                        preferred_element_type=jnp.float32)
    d2 = (na - 2.0 * g + nb) * (1.0 - ident)
    d = jnp.sqrt(jnp.maximum(d2, 1e-6))
    e = jnp.minimum(jnp.exp(-DIST_MULT * d), 1.0)
    e = jnp.maximum(e, 0.0)
    o_ref[0] = e * m * mrow


@functools.lru_cache(maxsize=1)
def _make_sc_scatter_a():
    info = plsc.get_sparse_core_info()
    nc = info.num_cores
    mesh = plsc.VectorSubcoreMesh(core_axis_name="c", subcore_axis_name="s")
    out_type = [
        jax.ShapeDtypeStruct((N_BATCH, N_POINTS, PAY_W), jnp.int32),
        jax.ShapeDtypeStruct((N_BATCH, N_POINTS, DIST_DIM), jnp.float32),
    ]
    scratch = [
        pltpu.VMEM((CHUNK,), jnp.int32),
        pltpu.VMEM((CHUNK, PAY_W), jnp.int32),
        pltpu.VMEM((CHUNK, DIST_DIM), jnp.float32),
        pltpu.SemaphoreType.DMA,
    ]

    @functools.partial(pl.kernel, mesh=mesh, out_type=out_type,
                       scratch_types=scratch)
    def scatter_a(rank_hbm, pay_hbm, xmsg_hbm,
                  pay_out, xmsgb_out,
                  idx_v, pay_v, xm_v, sem):
        wid = lax.axis_index("s") * nc + lax.axis_index("c")
        base = wid * CHUNK
        for b in range(N_BATCH):
            pltpu.sync_copy(rank_hbm.at[b].at[pl.ds(base, CHUNK)], idx_v)
            pltpu.sync_copy(pay_hbm.at[b].at[pl.ds(base, CHUNK), :], pay_v)
            pltpu.sync_copy(xmsg_hbm.at[b].at[pl.ds(base, CHUNK), :], xm_v)
            c1 = pltpu.async_copy(pay_v, pay_out.at[b].at[idx_v], sem)
            c2 = pltpu.async_copy(xm_v, xmsgb_out.at[b].at[idx_v], sem)
            c1.wait()
            c2.wait()

    return scatter_a


@functools.lru_cache(maxsize=1)
def _make_sc_scatter_b():
    info = plsc.get_sparse_core_info()
    nc = info.num_cores
    mesh = plsc.VectorSubcoreMesh(core_axis_name="c", subcore_axis_name="s")
    out_type = jax.ShapeDtypeStruct((N_BATCH, N_POINTS, NODE_DIM), jnp.float32)
    scratch = [
        pltpu.VMEM((CHUNK,), jnp.int32),
        pltpu.VMEM((CHUNK, NODE_DIM), jnp.float32),
        pltpu.SemaphoreType.DMA,
    ]

    @functools.partial(pl.kernel, mesh=mesh, out_type=out_type,
                       scratch_types=scratch)
    def scatter_b(rank_hbm, xnode_hbm, xnodeb_out, idx_v, xn_v, sem):
        wid = lax.axis_index("s") * nc + lax.axis_index("c")
        base = wid * CHUNK
        for b in range(N_BATCH):
            pltpu.sync_copy(rank_hbm.at[b].at[pl.ds(base, CHUNK)], idx_v)
            pltpu.sync_copy(xnode_hbm.at[b].at[pl.ds(base, CHUNK), :], xn_v)
            pltpu.async_copy(xn_v, xnodeb_out.at[b].at[idx_v], sem).wait()

    return scatter_b


def _rank_call(x_msg, w16, mskf_col):
    return pl.pallas_call(
        _rank_body,
        grid=(N_BATCH,),
        in_specs=[
            pl.BlockSpec((1, N_POINTS, DIST_DIM), lambda b: (b, 0, 0)),
            pl.BlockSpec((DIST_DIM, N_BINS // 2), lambda b: (0, 0)),
            pl.BlockSpec((1, N_POINTS, 1), lambda b: (b, 0, 0)),
        ],
        out_specs=pl.BlockSpec((1, N_POINTS, 1), lambda b: (b, 0, 0)),
        out_shape=jax.ShapeDtypeStruct((N_BATCH, N_POINTS, 1), jnp.int32),
    )(x_msg, w16, mskf_col)


def _dm_call(xb, mb, ident):
    return pl.pallas_call(
        _dm_body,
        grid=(N_BATCH * N_BINS,),
        in_specs=[
            pl.BlockSpec((1, BIN_SIZE, DIST_DIM), lambda i: (i, 0, 0)),
            pl.BlockSpec((1, BIN_SIZE, 1), lambda i: (i, 0, 0)),
            pl.BlockSpec((BIN_SIZE, BIN_SIZE), lambda i: (0, 0)),
        ],
        out_specs=pl.BlockSpec((1, BIN_SIZE, BIN_SIZE), lambda i: (i, 0, 0)),
        out_shape=jax.ShapeDtypeStruct(
            (N_BATCH * N_BINS, BIN_SIZE, BIN_SIZE), jnp.float32),
    )(xb, mb, ident)


@jax.jit
def kernel(x_msg, x_node, msk, W):
    mskf = msk.astype(jnp.float32)
    w16 = W[:, : N_BINS // 2]

    rank = _rank_call(x_msg, w16, mskf[..., None])
    rank2 = rank.reshape(N_BATCH, N_POINTS)

    packed = 2 * jnp.arange(N_POINTS, dtype=jnp.int32)[None, :] + msk.astype(
        jnp.int32)
    pay = jnp.broadcast_to(packed[..., None], (N_BATCH, N_POINTS, PAY_W))

    pay_out, xmsgb = _make_sc_scatter_a()(rank2, pay, x_msg)
    xnodeb = _make_sc_scatter_b()(rank2, x_node)

    packed_sorted = pay_out[..., 0]
    mskb_flat = (packed_sorted & 1).astype(jnp.float32)
    xb = xmsgb.reshape(N_BATCH * N_BINS, BIN_SIZE, DIST_DIM)
    mb = mskb_flat.reshape(N_BATCH * N_BINS, BIN_SIZE, 1)
    ident = jnp.eye(BIN_SIZE, dtype=jnp.float32)
    dm = _dm_call(xb, mb, ident)

    bins_split = (packed_sorted >> 1).reshape(N_BATCH, N_BINS, BIN_SIZE)
    x_features_binned = xnodeb.reshape(N_BATCH, N_BINS, BIN_SIZE, NODE_DIM)
    dm_out = dm.reshape(N_BATCH, N_BINS, BIN_SIZE, BIN_SIZE, 1)
    msk_f_binned = mskb_flat.reshape(N_BATCH, N_BINS, BIN_SIZE, 1)
    return (bins_split, x_features_binned, dm_out, msk_f_binned)
